# baseline (speedup 1.0000x reference)
import jax
import jax.numpy as jnp
from jax.experimental import pallas as pl
from jax.experimental.pallas import tpu as pltpu

_B = 8
_N = 200
_D = 64
_S = 64
_CH = 8
_NSYS = 10


def _gelu(x):
    return 0.5 * x * (1.0 + jax.lax.erf(x * 0.7071067811865476))


def _ln_rows(x, g, b):
    mu = jnp.mean(x, axis=-1, keepdims=True)
    v = jnp.mean((x - mu) ** 2, axis=-1, keepdims=True)
    return (x - mu) / jnp.sqrt(v + 1e-5) * g + b


def _dot(a, b):
    return jax.lax.dot_general(
        a, b, (((1,), (0,)), ((), ())), preferred_element_type=jnp.float32)


def _dot_t(a, b):
    return jax.lax.dot_general(
        a, b, (((0,), (0,)), ((), ())), preferred_element_type=jnp.float32)


_V_WNE, _V_BNE = 0, 1
_V_L0 = 2
_V_PER_L = 10
_V_BP = _V_L0 + 2 * _V_PER_L

_M_PER_L = 9
_M_WP1 = 2 * _M_PER_L


def _gnn_body(adj_ref, fs_ref, mats_ref, vecs_ref, seproj_ref, out_ref):
    a = adj_ref[0]
    mask = (a != 0.0).astype(jnp.float32)
    deg = jnp.sum(a, axis=-1, keepdims=True)
    cnt = jnp.sum(mask, axis=-1, keepdims=True)
    x = deg * vecs_ref[_V_WNE:_V_WNE + 1, :] + vecs_ref[_V_BNE:_V_BNE + 1, :]

    for l in range(2):
        mb = _M_PER_L * l
        vb = _V_L0 + _V_PER_L * l
        xn = _ln_rows(x, vecs_ref[vb:vb + 1, :], vecs_ref[vb + 1:vb + 2, :])
        xi = _dot(xn, mats_ref[mb]).astype(jnp.bfloat16)
        xj = (_dot(xn, mats_ref[mb + 1])
              + vecs_ref[vb + 2:vb + 3, :]).astype(jnp.bfloat16)
        maskb = mask.astype(jnp.bfloat16)
        s_parts = []
        for c in range(_N // _CH):
            xic = jax.lax.slice(xi, (c * _CH, 0), ((c + 1) * _CH, _D))
            mc = jax.lax.slice(maskb, (c * _CH, 0), ((c + 1) * _CH, _N))
            g = _gelu(xic[:, None, :] + xj[None, :, :])
            gm = g * mc[:, :, None]
            gm = gm[:, :100, :] + gm[:, 100:, :]
            gm = gm[:, :50, :] + gm[:, 50:, :]
            gm = gm[:, :25, :] + gm[:, 25:, :]
            s_parts.append(jnp.sum(gm.astype(jnp.float32), axis=1))
        s = jnp.concatenate(s_parts, axis=0)
        msum = _dot(s, mats_ref[mb + 2]) + cnt * vecs_ref[vb + 3:vb + 4, :]
        m = jnp.where(cnt > 0.0, msum / jnp.maximum(cnt, 1.0), 0.0)
        gir = _dot(m, mats_ref[mb + 3]) + vecs_ref[vb + 4:vb + 5, :]
        giz = _dot(m, mats_ref[mb + 4]) + vecs_ref[vb + 5:vb + 6, :]
        gin = _dot(m, mats_ref[mb + 5]) + vecs_ref[vb + 6:vb + 7, :]
        ghr = _dot(x, mats_ref[mb + 6]) + vecs_ref[vb + 7:vb + 8, :]
        ghz = _dot(x, mats_ref[mb + 7]) + vecs_ref[vb + 8:vb + 9, :]
        ghn = _dot(x, mats_ref[mb + 8]) + vecs_ref[vb + 9:vb + 10, :]
        r = jax.nn.sigmoid(gir + ghr)
        z = jax.nn.sigmoid(giz + ghz)
        n = jnp.tanh(gin + r * ghn)
        x = x + (1.0 - z) * n + z * x

    fsv = fs_ref[0]
    vio = jax.lax.broadcasted_iota(jnp.int32, (_NSYS, _N), 0)
    oht = (vio == fsv).astype(jnp.float32)
    less_t = (vio > fsv).astype(jnp.float32)
    sep = _dot_t(oht, seproj_ref[...])
    enh = _gelu(_dot(x, mats_ref[_M_WP1]) + sep
                + vecs_ref[_V_BP:_V_BP + 1, :])

    less = _dot_t(oht, less_t)
    eq = _dot_t(oht, oht)
    io_i = jax.lax.broadcasted_iota(jnp.int32, (_N, _N), 0)
    io_j = jax.lax.broadcasted_iota(jnp.int32, (_N, _N), 1)
    lt = (io_i > io_j).astype(jnp.float32)
    rank = jnp.sum(less + eq * lt, axis=-1, keepdims=True)
    perm_t = (rank.astype(jnp.int32) == io_j).astype(jnp.float32)
    out_ref[0] = _dot_t(perm_t, enh)


_SM_PER_L = 3
_SM_WR = 2 * _SM_PER_L
_SV_PER_L = 4
_SV_RG = 2 * _SV_PER_L


def _softplus(x):
    return jnp.maximum(x, 0.0) + jnp.log(1.0 + jnp.exp(-jnp.abs(x)))


def _ssm_body(xst_ref, smats_ref, amats_ref, svecs_ref, out_ref,
              dtd_s, bm_s, cm_s, dxd_s, ys_s):
    xx = xst_ref[...].reshape(_N * _B, _D)
    for l in range(2):
        vb = _SV_PER_L * l
        mb = _SM_PER_L * l
        xn2 = _ln_rows(xx, svecs_ref[vb:vb + 1, :], svecs_ref[vb + 1:vb + 2, :])
        dt = _softplus(_dot(xn2, smats_ref[mb]) + svecs_ref[vb + 2:vb + 3, :])
        dtd_s[...] = jnp.concatenate([dt, dt], axis=1)
        bm_s[...] = _dot(xn2, smats_ref[mb + 1])
        cm_s[...] = _dot(xn2, smats_ref[mb + 2])
        dx = dt * xn2
        dxd_s[...] = jnp.concatenate([dx, dx], axis=1)
        a2 = amats_ref[l]

        def step(t, hs):
            b0 = pl.multiple_of(t * _B, _B)
            dtd = dtd_s[pl.ds(b0, _B), :]
            dxd = dxd_s[pl.ds(b0, _B), :]
            btt = bm_s[pl.ds(b0, _B), :]
            ctt = cm_s[pl.ds(b0, _B), :]
            hs2 = []
            yacc = None
            for g in range(4):
                a2g = a2[8 * g:8 * g + 8]
                bsel = jnp.concatenate([
                    jnp.broadcast_to(btt[:, 8 * g:8 * g + 8][:, :, None],
                                     (_B, 8, _D)),
                    jnp.broadcast_to(btt[:, 32 + 8 * g:40 + 8 * g][:, :, None],
                                     (_B, 8, _D))], axis=2)
                csel = jnp.concatenate([
                    jnp.broadcast_to(ctt[:, 8 * g:8 * g + 8][:, :, None],
                                     (_B, 8, _D)),
                    jnp.broadcast_to(ctt[:, 32 + 8 * g:40 + 8 * g][:, :, None],
                                     (_B, 8, _D))], axis=2)
                da = jnp.exp(dtd[:, None, :] * a2g[None, :, :])
                h = da * hs[g] + dxd[:, None, :] * bsel
                hs2.append(h)
                yp = jnp.sum(h * csel, axis=1)
                yacc = yp if yacc is None else yacc + yp
            ys_s[pl.ds(b0, _B), :] = yacc[:, :_D] + yacc[:, _D:]
            return tuple(hs2)

        init = tuple(jnp.zeros((_B, 8, 2 * _D), jnp.float32) for _ in range(4))
        jax.lax.fori_loop(0, _N, step, init, unroll=2)
        xx = xx + ys_s[...] + xn2 * svecs_ref[vb + 3:vb + 4, :]

    xr = _ln_rows(xx, svecs_ref[_SV_RG:_SV_RG + 1, :],
                  svecs_ref[_SV_RG + 1:_SV_RG + 2, :])
    ne = _gelu(_dot(xr, smats_ref[_SM_WR])
               + svecs_ref[_SV_RG + 2:_SV_RG + 3, :])
    out_ref[...] = jnp.sum(ne.reshape(_N, _B, _D), axis=0) * (1.0 / _N)


def kernel(adj_matrix, functional_systems, params):
    p = params
    adj = adj_matrix.astype(jnp.float32)
    fs = functional_systems.astype(jnp.int32).reshape(_B, 1, _N)

    mats = []
    vecs = [p['W_ne'][0], p['b_ne']]
    for l in range(2):
        w1 = p['Wm1_%d' % l]
        wih = p['Wih_%d' % l]
        whh = p['Whh_%d' % l]
        mats += [w1[:_D], w1[_D:], p['Wm2_%d' % l],
                 wih[:, :_D], wih[:, _D:2 * _D], wih[:, 2 * _D:],
                 whh[:, :_D], whh[:, _D:2 * _D], whh[:, 2 * _D:]]
        bih = p['bih_%d' % l]
        bhh = p['bhh_%d' % l]
        vecs += [p['ln_g%d' % l], p['ln_b%d' % l],
                 p['bm1_%d' % l], p['bm2_%d' % l],
                 bih[:_D], bih[_D:2 * _D], bih[2 * _D:],
                 bhh[:_D], bhh[_D:2 * _D], bhh[2 * _D:]]
    mats.append(p['Wp'][:_D])
    vecs.append(p['bp'])
    mats = jnp.stack(mats)
    vecs = jnp.stack(vecs)
    seproj = p['sys_emb'] @ p['Wp'][_D:]

    xs = pl.pallas_call(
        _gnn_body,
        grid=(_B,),
        compiler_params=pltpu.CompilerParams(
            dimension_semantics=("parallel",)),
        in_specs=[
            pl.BlockSpec((1, _N, _N), lambda b: (b, 0, 0)),
            pl.BlockSpec((1, 1, _N), lambda b: (b, 0, 0)),
            pl.BlockSpec(mats.shape, lambda b: (0, 0, 0)),
            pl.BlockSpec(vecs.shape, lambda b: (0, 0)),
            pl.BlockSpec(seproj.shape, lambda b: (0, 0)),
        ],
        out_specs=pl.BlockSpec((1, _N, _D), lambda b: (b, 0, 0)),
        out_shape=jax.ShapeDtypeStruct((_B, _N, _D), jnp.float32),
    )(adj, fs, mats, vecs, seproj)

    xst = xs.transpose(1, 0, 2)

    smats = jnp.stack([p['Wdt_0'], p['WB_0'], p['WC_0'],
                       p['Wdt_1'], p['WB_1'], p['WC_1'], p['Wr']])
    at0 = -jnp.exp(p['Alog_0']).T
    at1 = -jnp.exp(p['Alog_1']).T
    amats = jnp.stack([
        jnp.concatenate([at0[:_S // 2], at0[_S // 2:]], axis=1),
        jnp.concatenate([at1[:_S // 2], at1[_S // 2:]], axis=1)])
    svecs = jnp.stack([p['sln_g0'], p['sln_b0'], p['bdt_0'], p['Dp_0'],
                       p['sln_g1'], p['sln_b1'], p['bdt_1'], p['Dp_1'],
                       p['rln_g'], p['rln_b'], p['br']])

    out = pl.pallas_call(
        _ssm_body,
        in_specs=[
            pl.BlockSpec(xst.shape, lambda: (0, 0, 0)),
            pl.BlockSpec(smats.shape, lambda: (0, 0, 0)),
            pl.BlockSpec(amats.shape, lambda: (0, 0, 0)),
            pl.BlockSpec(svecs.shape, lambda: (0, 0)),
        ],
        out_specs=pl.BlockSpec((_B, _D), lambda: (0, 0)),
        out_shape=jax.ShapeDtypeStruct((_B, _D), jnp.float32),
        scratch_shapes=[pltpu.VMEM((_N * _B, 2 * _D), jnp.float32),
                        pltpu.VMEM((_N * _B, _D), jnp.float32),
                        pltpu.VMEM((_N * _B, _D), jnp.float32),
                        pltpu.VMEM((_N * _B, 2 * _D), jnp.float32),
                        pltpu.VMEM((_N * _B, _D), jnp.float32)],
    )(xst, smats, amats, svecs)
    return out


# speedup vs baseline: 1.1791x; 1.1791x over previous
import jax
import jax.numpy as jnp
from jax.experimental import pallas as pl
from jax.experimental.pallas import tpu as pltpu

_B = 8
_N = 200
_D = 64
_S = 64
_CH = 8
_NSYS = 10


def _gelu(x):
    return 0.5 * x * (1.0 + jax.lax.erf(x * 0.7071067811865476))


def _ln_rows(x, g, b):
    mu = jnp.mean(x, axis=-1, keepdims=True)
    v = jnp.mean((x - mu) ** 2, axis=-1, keepdims=True)
    return (x - mu) / jnp.sqrt(v + 1e-5) * g + b


def _dot(a, b):
    return jax.lax.dot_general(
        a, b, (((1,), (0,)), ((), ())), preferred_element_type=jnp.float32)


def _dot_t(a, b):
    return jax.lax.dot_general(
        a, b, (((0,), (0,)), ((), ())), preferred_element_type=jnp.float32)


_V_WNE, _V_BNE = 0, 1
_V_L0 = 2
_V_PER_L = 10
_V_BP = _V_L0 + 2 * _V_PER_L

_M_PER_L = 9
_M_WP1 = 2 * _M_PER_L


def _gnn_body(adj_ref, fs_ref, mats_ref, vecs_ref, seproj_ref, out_ref):
    a = adj_ref[0]
    mask = (a != 0.0).astype(jnp.float32)
    deg = jnp.sum(a, axis=-1, keepdims=True)
    cnt = jnp.sum(mask, axis=-1, keepdims=True)
    x = deg * vecs_ref[_V_WNE:_V_WNE + 1, :] + vecs_ref[_V_BNE:_V_BNE + 1, :]

    for l in range(2):
        mb = _M_PER_L * l
        vb = _V_L0 + _V_PER_L * l
        xn = _ln_rows(x, vecs_ref[vb:vb + 1, :], vecs_ref[vb + 1:vb + 2, :])
        xi = _dot(xn, mats_ref[mb]).astype(jnp.bfloat16)
        xj = (_dot(xn, mats_ref[mb + 1])
              + vecs_ref[vb + 2:vb + 3, :]).astype(jnp.bfloat16)
        maskb = mask.astype(jnp.bfloat16)
        s_parts = []
        for c in range(_N // _CH):
            xic = jax.lax.slice(xi, (c * _CH, 0), ((c + 1) * _CH, _D))
            mc = jax.lax.slice(maskb, (c * _CH, 0), ((c + 1) * _CH, _N))
            g = _gelu(xic[:, None, :] + xj[None, :, :])
            gm = g * mc[:, :, None]
            gm = gm[:, :100, :] + gm[:, 100:, :]
            gm = gm[:, :50, :] + gm[:, 50:, :]
            gm = gm[:, :25, :] + gm[:, 25:, :]
            s_parts.append(jnp.sum(gm.astype(jnp.float32), axis=1))
        s = jnp.concatenate(s_parts, axis=0)
        msum = _dot(s, mats_ref[mb + 2]) + cnt * vecs_ref[vb + 3:vb + 4, :]
        m = jnp.where(cnt > 0.0, msum / jnp.maximum(cnt, 1.0), 0.0)
        gir = _dot(m, mats_ref[mb + 3]) + vecs_ref[vb + 4:vb + 5, :]
        giz = _dot(m, mats_ref[mb + 4]) + vecs_ref[vb + 5:vb + 6, :]
        gin = _dot(m, mats_ref[mb + 5]) + vecs_ref[vb + 6:vb + 7, :]
        ghr = _dot(x, mats_ref[mb + 6]) + vecs_ref[vb + 7:vb + 8, :]
        ghz = _dot(x, mats_ref[mb + 7]) + vecs_ref[vb + 8:vb + 9, :]
        ghn = _dot(x, mats_ref[mb + 8]) + vecs_ref[vb + 9:vb + 10, :]
        r = jax.nn.sigmoid(gir + ghr)
        z = jax.nn.sigmoid(giz + ghz)
        n = jnp.tanh(gin + r * ghn)
        x = x + (1.0 - z) * n + z * x

    fsv = fs_ref[0]
    vio = jax.lax.broadcasted_iota(jnp.int32, (_NSYS, _N), 0)
    oht = (vio == fsv).astype(jnp.float32)
    less_t = (vio > fsv).astype(jnp.float32)
    sep = _dot_t(oht, seproj_ref[...])
    enh = _gelu(_dot(x, mats_ref[_M_WP1]) + sep
                + vecs_ref[_V_BP:_V_BP + 1, :])

    less = _dot_t(oht, less_t)
    eq = _dot_t(oht, oht)
    io_i = jax.lax.broadcasted_iota(jnp.int32, (_N, _N), 0)
    io_j = jax.lax.broadcasted_iota(jnp.int32, (_N, _N), 1)
    lt = (io_i > io_j).astype(jnp.float32)
    rank = jnp.sum(less + eq * lt, axis=-1, keepdims=True)
    perm_t = (rank.astype(jnp.int32) == io_j).astype(jnp.float32)
    out_ref[0] = _dot_t(perm_t, enh)


_SM_PER_L = 3
_SM_WR = 2 * _SM_PER_L
_SV_PER_L = 4
_SV_RG = 2 * _SV_PER_L


def _softplus(x):
    return jnp.maximum(x, 0.0) + jnp.log(1.0 + jnp.exp(-jnp.abs(x)))


def _ssm_body(xst_ref, smats_ref, amats_ref, svecs_ref, m2_ref, out_ref,
              dtd_s, bsel_s, csel_s, dxd_s, ys_s):
    xx = xst_ref[...].reshape(_N * _B, _D)
    for l in range(2):
        vb = _SV_PER_L * l
        mb = _SM_PER_L * l
        xn2 = _ln_rows(xx, svecs_ref[vb:vb + 1, :], svecs_ref[vb + 1:vb + 2, :])
        dt = _softplus(_dot(xn2, smats_ref[mb]) + svecs_ref[vb + 2:vb + 3, :])
        dtd_s[...] = jnp.concatenate([dt, dt], axis=1)
        dx = dt * xn2
        dxd_s[...] = jnp.concatenate([dx, dx], axis=1)
        bm = _dot(xn2, smats_ref[mb + 1])
        cm = _dot(xn2, smats_ref[mb + 2])
        bsel_s[...] = jax.lax.dot_general(
            bm.astype(jnp.bfloat16), m2_ref[...], (((1,), (0,)), ((), ())),
            preferred_element_type=jnp.float32).astype(
                jnp.bfloat16).reshape(_N * _B, 32, 2 * _D)
        csel_s[...] = jax.lax.dot_general(
            cm.astype(jnp.bfloat16), m2_ref[...], (((1,), (0,)), ((), ())),
            preferred_element_type=jnp.float32).astype(
                jnp.bfloat16).reshape(_N * _B, 32, 2 * _D)
        a2 = amats_ref[l]

        def step(t, hs):
            b0 = pl.multiple_of(t * _B, _B)
            dtd = dtd_s[pl.ds(b0, _B), :]
            dxd = dxd_s[pl.ds(b0, _B), :]
            hs2 = []
            yg = []
            for g in range(4):
                bsel = bsel_s[pl.ds(b0 + 2 * g, 2)].astype(jnp.float32)
                csel = csel_s[pl.ds(b0 + 2 * g, 2)].astype(jnp.float32)
                da = jnp.exp(dtd[2 * g:2 * g + 2][:, None, :]
                             * a2[None, :, :])
                h = da * hs[g] + dxd[2 * g:2 * g + 2][:, None, :] * bsel
                hs2.append(h)
                yg.append(jnp.sum(h * csel, axis=1))
            yacc = jnp.concatenate(yg, axis=0)
            ys_s[pl.ds(b0, _B), :] = yacc[:, :_D] + yacc[:, _D:]
            return tuple(hs2)

        init = tuple(jnp.zeros((2, 32, 2 * _D), jnp.float32) for _ in range(4))
        jax.lax.fori_loop(0, _N, step, init, unroll=2)
        xx = xx + ys_s[...] + xn2 * svecs_ref[vb + 3:vb + 4, :]

    xr = _ln_rows(xx, svecs_ref[_SV_RG:_SV_RG + 1, :],
                  svecs_ref[_SV_RG + 1:_SV_RG + 2, :])
    ne = _gelu(_dot(xr, smats_ref[_SM_WR])
               + svecs_ref[_SV_RG + 2:_SV_RG + 3, :])
    out_ref[...] = jnp.sum(ne.reshape(_N, _B, _D), axis=0) * (1.0 / _N)


def kernel(adj_matrix, functional_systems, params):
    p = params
    adj = adj_matrix.astype(jnp.float32)
    fs = functional_systems.astype(jnp.int32).reshape(_B, 1, _N)

    mats = []
    vecs = [p['W_ne'][0], p['b_ne']]
    for l in range(2):
        w1 = p['Wm1_%d' % l]
        wih = p['Wih_%d' % l]
        whh = p['Whh_%d' % l]
        mats += [w1[:_D], w1[_D:], p['Wm2_%d' % l],
                 wih[:, :_D], wih[:, _D:2 * _D], wih[:, 2 * _D:],
                 whh[:, :_D], whh[:, _D:2 * _D], whh[:, 2 * _D:]]
        bih = p['bih_%d' % l]
        bhh = p['bhh_%d' % l]
        vecs += [p['ln_g%d' % l], p['ln_b%d' % l],
                 p['bm1_%d' % l], p['bm2_%d' % l],
                 bih[:_D], bih[_D:2 * _D], bih[2 * _D:],
                 bhh[:_D], bhh[_D:2 * _D], bhh[2 * _D:]]
    mats.append(p['Wp'][:_D])
    vecs.append(p['bp'])
    mats = jnp.stack(mats)
    vecs = jnp.stack(vecs)
    seproj = p['sys_emb'] @ p['Wp'][_D:]

    xs = pl.pallas_call(
        _gnn_body,
        grid=(_B,),
        compiler_params=pltpu.CompilerParams(
            dimension_semantics=("parallel",)),
        in_specs=[
            pl.BlockSpec((1, _N, _N), lambda b: (b, 0, 0)),
            pl.BlockSpec((1, 1, _N), lambda b: (b, 0, 0)),
            pl.BlockSpec(mats.shape, lambda b: (0, 0, 0)),
            pl.BlockSpec(vecs.shape, lambda b: (0, 0)),
            pl.BlockSpec(seproj.shape, lambda b: (0, 0)),
        ],
        out_specs=pl.BlockSpec((1, _N, _D), lambda b: (b, 0, 0)),
        out_shape=jax.ShapeDtypeStruct((_B, _N, _D), jnp.float32),
    )(adj, fs, mats, vecs, seproj)

    xst = xs.transpose(1, 0, 2)

    smats = jnp.stack([p['Wdt_0'], p['WB_0'], p['WC_0'],
                       p['Wdt_1'], p['WB_1'], p['WC_1'], p['Wr']])
    at0 = -jnp.exp(p['Alog_0']).T
    at1 = -jnp.exp(p['Alog_1']).T
    amats = jnp.stack([
        jnp.concatenate([at0[:_S // 2], at0[_S // 2:]], axis=1),
        jnp.concatenate([at1[:_S // 2], at1[_S // 2:]], axis=1)])
    svecs = jnp.stack([p['sln_g0'], p['sln_b0'], p['bdt_0'], p['Dp_0'],
                       p['sln_g1'], p['sln_b1'], p['bdt_1'], p['Dp_1'],
                       p['rln_g'], p['rln_b'], p['br']])

    col = jnp.arange(32 * 2 * _D)[None, :]
    m2 = (jnp.arange(_S)[:, None]
          == col // (2 * _D) + 32 * ((col % (2 * _D)) >= _D))
    m2 = m2.astype(jnp.bfloat16)

    out = pl.pallas_call(
        _ssm_body,
        in_specs=[
            pl.BlockSpec(xst.shape, lambda: (0, 0, 0)),
            pl.BlockSpec(smats.shape, lambda: (0, 0, 0)),
            pl.BlockSpec(amats.shape, lambda: (0, 0, 0)),
            pl.BlockSpec(svecs.shape, lambda: (0, 0)),
            pl.BlockSpec(m2.shape, lambda: (0, 0)),
        ],
        out_specs=pl.BlockSpec((_B, _D), lambda: (0, 0)),
        out_shape=jax.ShapeDtypeStruct((_B, _D), jnp.float32),
        scratch_shapes=[pltpu.VMEM((_N * _B, 2 * _D), jnp.float32),
                        pltpu.VMEM((_N * _B, 32, 2 * _D), jnp.bfloat16),
                        pltpu.VMEM((_N * _B, 32, 2 * _D), jnp.bfloat16),
                        pltpu.VMEM((_N * _B, 2 * _D), jnp.float32),
                        pltpu.VMEM((_N * _B, _D), jnp.float32)],
    )(xst, smats, amats, svecs, m2)
    return out


# speedup vs baseline: 1.2347x; 1.0471x over previous
import jax
import jax.numpy as jnp
from jax.experimental import pallas as pl
from jax.experimental.pallas import tpu as pltpu

_B = 8
_N = 200
_D = 64
_S = 64
_CH = 8
_NSYS = 10


def _gelu(x):
    return 0.5 * x * (1.0 + jax.lax.erf(x * 0.7071067811865476))


def _ln_rows(x, g, b):
    mu = jnp.mean(x, axis=-1, keepdims=True)
    v = jnp.mean((x - mu) ** 2, axis=-1, keepdims=True)
    return (x - mu) / jnp.sqrt(v + 1e-5) * g + b


def _dot(a, b):
    return jax.lax.dot_general(
        a, b, (((1,), (0,)), ((), ())), preferred_element_type=jnp.float32)


def _dot_t(a, b):
    return jax.lax.dot_general(
        a, b, (((0,), (0,)), ((), ())), preferred_element_type=jnp.float32)


_V_WNE, _V_BNE = 0, 1
_V_L0 = 2
_V_PER_L = 10
_V_BP = _V_L0 + 2 * _V_PER_L

_M_PER_L = 9
_M_WP1 = 2 * _M_PER_L


def _gnn_body(adj_ref, fs_ref, mats_ref, vecs_ref, seproj_ref, out_ref):
    a = adj_ref[0]
    mask = (a != 0.0).astype(jnp.float32)
    deg = jnp.sum(a, axis=-1, keepdims=True)
    cnt = jnp.sum(mask, axis=-1, keepdims=True)
    x = deg * vecs_ref[_V_WNE:_V_WNE + 1, :] + vecs_ref[_V_BNE:_V_BNE + 1, :]

    for l in range(2):
        mb = _M_PER_L * l
        vb = _V_L0 + _V_PER_L * l
        xn = _ln_rows(x, vecs_ref[vb:vb + 1, :], vecs_ref[vb + 1:vb + 2, :])
        xi = _dot(xn, mats_ref[mb]).astype(jnp.bfloat16)
        xj = (_dot(xn, mats_ref[mb + 1])
              + vecs_ref[vb + 2:vb + 3, :]).astype(jnp.bfloat16)
        maskb = mask.astype(jnp.bfloat16)
        s_parts = []
        for c in range(_N // _CH):
            xic = jax.lax.slice(xi, (c * _CH, 0), ((c + 1) * _CH, _D))
            mc = jax.lax.slice(maskb, (c * _CH, 0), ((c + 1) * _CH, _N))
            g = _gelu(xic[:, None, :] + xj[None, :, :])
            gm = g * mc[:, :, None]
            gm = gm[:, :100, :] + gm[:, 100:, :]
            gm = gm[:, :50, :] + gm[:, 50:, :]
            gm = gm[:, :25, :] + gm[:, 25:, :]
            s_parts.append(jnp.sum(gm.astype(jnp.float32), axis=1))
        s = jnp.concatenate(s_parts, axis=0)
        msum = _dot(s, mats_ref[mb + 2]) + cnt * vecs_ref[vb + 3:vb + 4, :]
        m = jnp.where(cnt > 0.0, msum / jnp.maximum(cnt, 1.0), 0.0)
        gir = _dot(m, mats_ref[mb + 3]) + vecs_ref[vb + 4:vb + 5, :]
        giz = _dot(m, mats_ref[mb + 4]) + vecs_ref[vb + 5:vb + 6, :]
        gin = _dot(m, mats_ref[mb + 5]) + vecs_ref[vb + 6:vb + 7, :]
        ghr = _dot(x, mats_ref[mb + 6]) + vecs_ref[vb + 7:vb + 8, :]
        ghz = _dot(x, mats_ref[mb + 7]) + vecs_ref[vb + 8:vb + 9, :]
        ghn = _dot(x, mats_ref[mb + 8]) + vecs_ref[vb + 9:vb + 10, :]
        r = jax.nn.sigmoid(gir + ghr)
        z = jax.nn.sigmoid(giz + ghz)
        n = jnp.tanh(gin + r * ghn)
        x = x + (1.0 - z) * n + z * x

    fsv = fs_ref[0]
    vio = jax.lax.broadcasted_iota(jnp.int32, (_NSYS, _N), 0)
    oht = (vio == fsv).astype(jnp.float32)
    less_t = (vio > fsv).astype(jnp.float32)
    sep = _dot_t(oht, seproj_ref[...])
    enh = _gelu(_dot(x, mats_ref[_M_WP1]) + sep
                + vecs_ref[_V_BP:_V_BP + 1, :])

    less = _dot_t(oht, less_t)
    eq = _dot_t(oht, oht)
    io_i = jax.lax.broadcasted_iota(jnp.int32, (_N, _N), 0)
    io_j = jax.lax.broadcasted_iota(jnp.int32, (_N, _N), 1)
    lt = (io_i > io_j).astype(jnp.float32)
    rank = jnp.sum(less + eq * lt, axis=-1, keepdims=True)
    perm_t = (rank.astype(jnp.int32) == io_j).astype(jnp.float32)
    out_ref[0] = _dot_t(perm_t, enh)


_SM_PER_L = 3
_SM_WR = 2 * _SM_PER_L
_SV_PER_L = 4
_SV_RG = 2 * _SV_PER_L


def _softplus(x):
    return jnp.maximum(x, 0.0) + jnp.log(1.0 + jnp.exp(-jnp.abs(x)))


def _ssm_body(xst_ref, smats_ref, amats_ref, svecs_ref, m2_ref, out_ref,
              dtd_s, bsel_s, csel_s, dxd_s, ys_s):
    xx = xst_ref[...].reshape(_N * _B, _D)
    for l in range(2):
        vb = _SV_PER_L * l
        mb = _SM_PER_L * l
        xn2 = _ln_rows(xx, svecs_ref[vb:vb + 1, :], svecs_ref[vb + 1:vb + 2, :])
        dt = _softplus(_dot(xn2, smats_ref[mb]) + svecs_ref[vb + 2:vb + 3, :])
        dtd_s[...] = jnp.concatenate([dt, dt], axis=1)
        dx = dt * xn2
        dxd_s[...] = jnp.concatenate([dx, dx], axis=1)
        bm = _dot(xn2, smats_ref[mb + 1])
        cm = _dot(xn2, smats_ref[mb + 2])
        bsel_s[...] = jax.lax.dot_general(
            bm.astype(jnp.bfloat16), m2_ref[...], (((1,), (0,)), ((), ())),
            preferred_element_type=jnp.float32).astype(
                jnp.bfloat16).reshape(_N * _B, 32, 2 * _D)
        csel_s[...] = jax.lax.dot_general(
            cm.astype(jnp.bfloat16), m2_ref[...], (((1,), (0,)), ((), ())),
            preferred_element_type=jnp.float32).astype(
                jnp.bfloat16).reshape(_N * _B, 32, 2 * _D)
        a2 = amats_ref[l]

        def step(t, hs):
            b0 = pl.multiple_of(t * _B, _B)
            dtd = dtd_s[pl.ds(b0, _B), :]
            dxd = dxd_s[pl.ds(b0, _B), :]
            hs2 = []
            yg = []
            for g in range(4):
                bsel = bsel_s[pl.ds(b0 + 2 * g, 2)].astype(jnp.float32)
                csel = csel_s[pl.ds(b0 + 2 * g, 2)].astype(jnp.float32)
                da = jnp.exp(dtd[2 * g:2 * g + 2][:, None, :]
                             * a2[None, :, :])
                h = da * hs[g] + dxd[2 * g:2 * g + 2][:, None, :] * bsel
                hs2.append(h)
                yg.append(jnp.sum(h * csel, axis=1))
            yacc = jnp.concatenate(yg, axis=0)
            ys_s[pl.ds(b0, _B), :] = yacc[:, :_D] + yacc[:, _D:]
            return tuple(hs2)

        init = tuple(jnp.zeros((2, 32, 2 * _D), jnp.float32) for _ in range(4))
        jax.lax.fori_loop(0, _N, step, init, unroll=4)
        xx = xx + ys_s[...] + xn2 * svecs_ref[vb + 3:vb + 4, :]

    xr = _ln_rows(xx, svecs_ref[_SV_RG:_SV_RG + 1, :],
                  svecs_ref[_SV_RG + 1:_SV_RG + 2, :])
    ne = _gelu(_dot(xr, smats_ref[_SM_WR])
               + svecs_ref[_SV_RG + 2:_SV_RG + 3, :])
    out_ref[...] = jnp.sum(ne.reshape(_N, _B, _D), axis=0) * (1.0 / _N)


def kernel(adj_matrix, functional_systems, params):
    p = params
    adj = adj_matrix.astype(jnp.float32)
    fs = functional_systems.astype(jnp.int32).reshape(_B, 1, _N)

    mats = []
    vecs = [p['W_ne'][0], p['b_ne']]
    for l in range(2):
        w1 = p['Wm1_%d' % l]
        wih = p['Wih_%d' % l]
        whh = p['Whh_%d' % l]
        mats += [w1[:_D], w1[_D:], p['Wm2_%d' % l],
                 wih[:, :_D], wih[:, _D:2 * _D], wih[:, 2 * _D:],
                 whh[:, :_D], whh[:, _D:2 * _D], whh[:, 2 * _D:]]
        bih = p['bih_%d' % l]
        bhh = p['bhh_%d' % l]
        vecs += [p['ln_g%d' % l], p['ln_b%d' % l],
                 p['bm1_%d' % l], p['bm2_%d' % l],
                 bih[:_D], bih[_D:2 * _D], bih[2 * _D:],
                 bhh[:_D], bhh[_D:2 * _D], bhh[2 * _D:]]
    mats.append(p['Wp'][:_D])
    vecs.append(p['bp'])
    mats = jnp.stack(mats)
    vecs = jnp.stack(vecs)
    seproj = p['sys_emb'] @ p['Wp'][_D:]

    xs = pl.pallas_call(
        _gnn_body,
        grid=(_B,),
        compiler_params=pltpu.CompilerParams(
            dimension_semantics=("parallel",)),
        in_specs=[
            pl.BlockSpec((1, _N, _N), lambda b: (b, 0, 0)),
            pl.BlockSpec((1, 1, _N), lambda b: (b, 0, 0)),
            pl.BlockSpec(mats.shape, lambda b: (0, 0, 0)),
            pl.BlockSpec(vecs.shape, lambda b: (0, 0)),
            pl.BlockSpec(seproj.shape, lambda b: (0, 0)),
        ],
        out_specs=pl.BlockSpec((1, _N, _D), lambda b: (b, 0, 0)),
        out_shape=jax.ShapeDtypeStruct((_B, _N, _D), jnp.float32),
    )(adj, fs, mats, vecs, seproj)

    xst = xs.transpose(1, 0, 2)

    smats = jnp.stack([p['Wdt_0'], p['WB_0'], p['WC_0'],
                       p['Wdt_1'], p['WB_1'], p['WC_1'], p['Wr']])
    at0 = -jnp.exp(p['Alog_0']).T
    at1 = -jnp.exp(p['Alog_1']).T
    amats = jnp.stack([
        jnp.concatenate([at0[:_S // 2], at0[_S // 2:]], axis=1),
        jnp.concatenate([at1[:_S // 2], at1[_S // 2:]], axis=1)])
    svecs = jnp.stack([p['sln_g0'], p['sln_b0'], p['bdt_0'], p['Dp_0'],
                       p['sln_g1'], p['sln_b1'], p['bdt_1'], p['Dp_1'],
                       p['rln_g'], p['rln_b'], p['br']])

    col = jnp.arange(32 * 2 * _D)[None, :]
    m2 = (jnp.arange(_S)[:, None]
          == col // (2 * _D) + 32 * ((col % (2 * _D)) >= _D))
    m2 = m2.astype(jnp.bfloat16)

    out = pl.pallas_call(
        _ssm_body,
        in_specs=[
            pl.BlockSpec(xst.shape, lambda: (0, 0, 0)),
            pl.BlockSpec(smats.shape, lambda: (0, 0, 0)),
            pl.BlockSpec(amats.shape, lambda: (0, 0, 0)),
            pl.BlockSpec(svecs.shape, lambda: (0, 0)),
            pl.BlockSpec(m2.shape, lambda: (0, 0)),
        ],
        out_specs=pl.BlockSpec((_B, _D), lambda: (0, 0)),
        out_shape=jax.ShapeDtypeStruct((_B, _D), jnp.float32),
        scratch_shapes=[pltpu.VMEM((_N * _B, 2 * _D), jnp.float32),
                        pltpu.VMEM((_N * _B, 32, 2 * _D), jnp.bfloat16),
                        pltpu.VMEM((_N * _B, 32, 2 * _D), jnp.bfloat16),
                        pltpu.VMEM((_N * _B, 2 * _D), jnp.float32),
                        pltpu.VMEM((_N * _B, _D), jnp.float32)],
    )(xst, smats, amats, svecs, m2)
    return out


# speedup vs baseline: 1.2598x; 1.0203x over previous
import jax
import jax.numpy as jnp
from jax.experimental import pallas as pl
from jax.experimental.pallas import tpu as pltpu

_B = 8
_N = 200
_D = 64
_S = 64
_CH = 8
_NSYS = 10


def _gelu(x):
    return 0.5 * x * (1.0 + jax.lax.erf(x * 0.7071067811865476))


def _ln_rows(x, g, b):
    mu = jnp.mean(x, axis=-1, keepdims=True)
    v = jnp.mean((x - mu) ** 2, axis=-1, keepdims=True)
    return (x - mu) / jnp.sqrt(v + 1e-5) * g + b


def _dot(a, b):
    return jax.lax.dot_general(
        a, b, (((1,), (0,)), ((), ())), preferred_element_type=jnp.float32)


def _dot_t(a, b):
    return jax.lax.dot_general(
        a, b, (((0,), (0,)), ((), ())), preferred_element_type=jnp.float32)


_V_WNE, _V_BNE = 0, 1
_V_L0 = 2
_V_PER_L = 10
_V_BP = _V_L0 + 2 * _V_PER_L

_M_PER_L = 9
_M_WP1 = 2 * _M_PER_L


def _gnn_body(adj_ref, fs_ref, mats_ref, vecs_ref, seproj_ref, out_ref):
    a = adj_ref[0]
    mask = (a != 0.0).astype(jnp.float32)
    deg = jnp.sum(a, axis=-1, keepdims=True)
    cnt = jnp.sum(mask, axis=-1, keepdims=True)
    x = deg * vecs_ref[_V_WNE:_V_WNE + 1, :] + vecs_ref[_V_BNE:_V_BNE + 1, :]

    for l in range(2):
        mb = _M_PER_L * l
        vb = _V_L0 + _V_PER_L * l
        xn = _ln_rows(x, vecs_ref[vb:vb + 1, :], vecs_ref[vb + 1:vb + 2, :])
        xi = _dot(xn, mats_ref[mb]).astype(jnp.bfloat16)
        xj = (_dot(xn, mats_ref[mb + 1])
              + vecs_ref[vb + 2:vb + 3, :]).astype(jnp.bfloat16)
        maskb = mask.astype(jnp.bfloat16)
        s_parts = []
        for c in range(_N // _CH):
            xic = jax.lax.slice(xi, (c * _CH, 0), ((c + 1) * _CH, _D))
            mc = jax.lax.slice(maskb, (c * _CH, 0), ((c + 1) * _CH, _N))
            g = _gelu(xic[:, None, :] + xj[None, :, :])
            gm = g * mc[:, :, None]
            gm = gm[:, :100, :] + gm[:, 100:, :]
            gm = gm[:, :50, :] + gm[:, 50:, :]
            gm = gm[:, :25, :] + gm[:, 25:, :]
            s_parts.append(jnp.sum(gm.astype(jnp.float32), axis=1))
        s = jnp.concatenate(s_parts, axis=0)
        msum = _dot(s, mats_ref[mb + 2]) + cnt * vecs_ref[vb + 3:vb + 4, :]
        m = jnp.where(cnt > 0.0, msum / jnp.maximum(cnt, 1.0), 0.0)
        gir = _dot(m, mats_ref[mb + 3]) + vecs_ref[vb + 4:vb + 5, :]
        giz = _dot(m, mats_ref[mb + 4]) + vecs_ref[vb + 5:vb + 6, :]
        gin = _dot(m, mats_ref[mb + 5]) + vecs_ref[vb + 6:vb + 7, :]
        ghr = _dot(x, mats_ref[mb + 6]) + vecs_ref[vb + 7:vb + 8, :]
        ghz = _dot(x, mats_ref[mb + 7]) + vecs_ref[vb + 8:vb + 9, :]
        ghn = _dot(x, mats_ref[mb + 8]) + vecs_ref[vb + 9:vb + 10, :]
        r = jax.nn.sigmoid(gir + ghr)
        z = jax.nn.sigmoid(giz + ghz)
        n = jnp.tanh(gin + r * ghn)
        x = x + (1.0 - z) * n + z * x

    fsv = fs_ref[0]
    vio = jax.lax.broadcasted_iota(jnp.int32, (_NSYS, _N), 0)
    oht = (vio == fsv).astype(jnp.float32)
    less_t = (vio > fsv).astype(jnp.float32)
    sep = _dot_t(oht, seproj_ref[...])
    enh = _gelu(_dot(x, mats_ref[_M_WP1]) + sep
                + vecs_ref[_V_BP:_V_BP + 1, :])

    less = _dot_t(oht, less_t)
    eq = _dot_t(oht, oht)
    io_i = jax.lax.broadcasted_iota(jnp.int32, (_N, _N), 0)
    io_j = jax.lax.broadcasted_iota(jnp.int32, (_N, _N), 1)
    lt = (io_i > io_j).astype(jnp.float32)
    rank = jnp.sum(less + eq * lt, axis=-1, keepdims=True)
    perm_t = (rank.astype(jnp.int32) == io_j).astype(jnp.float32)
    out_ref[0] = _dot_t(perm_t, enh)


_SM_PER_L = 3
_SM_WR = 2 * _SM_PER_L
_SV_PER_L = 4
_SV_RG = 2 * _SV_PER_L


def _softplus(x):
    return jnp.maximum(x, 0.0) + jnp.log(1.0 + jnp.exp(-jnp.abs(x)))


def _ssm_body(xst_ref, smats_ref, amats_ref, svecs_ref, m2_ref, out_ref,
              dtd_s, bsel_s, csel_s, dxd_s, ys_s):
    xx = xst_ref[...].reshape(_N * _B, _D)
    for l in range(2):
        vb = _SV_PER_L * l
        mb = _SM_PER_L * l
        xn2 = _ln_rows(xx, svecs_ref[vb:vb + 1, :], svecs_ref[vb + 1:vb + 2, :])
        dt = _softplus(_dot(xn2, smats_ref[mb]) + svecs_ref[vb + 2:vb + 3, :])
        dtd_s[...] = jnp.concatenate([dt, dt], axis=1)
        dx = dt * xn2
        dxd_s[...] = jnp.concatenate([dx, dx], axis=1)
        bm = _dot(xn2, smats_ref[mb + 1])
        cm = _dot(xn2, smats_ref[mb + 2])
        bsel_s[...] = jax.lax.dot_general(
            bm.astype(jnp.bfloat16), m2_ref[...], (((1,), (0,)), ((), ())),
            preferred_element_type=jnp.float32).astype(
                jnp.bfloat16).reshape(_N * _B, 32, 2 * _D)
        csel_s[...] = jax.lax.dot_general(
            cm.astype(jnp.bfloat16), m2_ref[...], (((1,), (0,)), ((), ())),
            preferred_element_type=jnp.float32).astype(
                jnp.bfloat16).reshape(_N * _B, 32, 2 * _D)
        a2 = amats_ref[l]

        def step(t, hs):
            b0 = pl.multiple_of(t * _B, _B)
            dtd = dtd_s[pl.ds(b0, _B), :]
            dxd = dxd_s[pl.ds(b0, _B), :]
            hs2 = []
            yg = []
            for g in range(4):
                bsel = bsel_s[pl.ds(b0 + 2 * g, 2)].astype(jnp.float32)
                csel = csel_s[pl.ds(b0 + 2 * g, 2)].astype(jnp.float32)
                da = jnp.exp(dtd[2 * g:2 * g + 2][:, None, :]
                             * a2[None, :, :])
                h = da * hs[g] + dxd[2 * g:2 * g + 2][:, None, :] * bsel
                hs2.append(h)
                yg.append(jnp.sum(h * csel, axis=1))
            yacc = jnp.concatenate(yg, axis=0)
            ys_s[pl.ds(b0, _B), :] = yacc[:, :_D] + yacc[:, _D:]
            return tuple(hs2)

        init = tuple(jnp.zeros((2, 32, 2 * _D), jnp.float32) for _ in range(4))
        jax.lax.fori_loop(0, _N, step, init, unroll=8)
        xx = xx + ys_s[...] + xn2 * svecs_ref[vb + 3:vb + 4, :]

    xr = _ln_rows(xx, svecs_ref[_SV_RG:_SV_RG + 1, :],
                  svecs_ref[_SV_RG + 1:_SV_RG + 2, :])
    ne = _gelu(_dot(xr, smats_ref[_SM_WR])
               + svecs_ref[_SV_RG + 2:_SV_RG + 3, :])
    out_ref[...] = jnp.sum(ne.reshape(_N, _B, _D), axis=0) * (1.0 / _N)


def kernel(adj_matrix, functional_systems, params):
    p = params
    adj = adj_matrix.astype(jnp.float32)
    fs = functional_systems.astype(jnp.int32).reshape(_B, 1, _N)

    mats = []
    vecs = [p['W_ne'][0], p['b_ne']]
    for l in range(2):
        w1 = p['Wm1_%d' % l]
        wih = p['Wih_%d' % l]
        whh = p['Whh_%d' % l]
        mats += [w1[:_D], w1[_D:], p['Wm2_%d' % l],
                 wih[:, :_D], wih[:, _D:2 * _D], wih[:, 2 * _D:],
                 whh[:, :_D], whh[:, _D:2 * _D], whh[:, 2 * _D:]]
        bih = p['bih_%d' % l]
        bhh = p['bhh_%d' % l]
        vecs += [p['ln_g%d' % l], p['ln_b%d' % l],
                 p['bm1_%d' % l], p['bm2_%d' % l],
                 bih[:_D], bih[_D:2 * _D], bih[2 * _D:],
                 bhh[:_D], bhh[_D:2 * _D], bhh[2 * _D:]]
    mats.append(p['Wp'][:_D])
    vecs.append(p['bp'])
    mats = jnp.stack(mats)
    vecs = jnp.stack(vecs)
    seproj = p['sys_emb'] @ p['Wp'][_D:]

    xs = pl.pallas_call(
        _gnn_body,
        grid=(_B,),
        compiler_params=pltpu.CompilerParams(
            dimension_semantics=("parallel",)),
        in_specs=[
            pl.BlockSpec((1, _N, _N), lambda b: (b, 0, 0)),
            pl.BlockSpec((1, 1, _N), lambda b: (b, 0, 0)),
            pl.BlockSpec(mats.shape, lambda b: (0, 0, 0)),
            pl.BlockSpec(vecs.shape, lambda b: (0, 0)),
            pl.BlockSpec(seproj.shape, lambda b: (0, 0)),
        ],
        out_specs=pl.BlockSpec((1, _N, _D), lambda b: (b, 0, 0)),
        out_shape=jax.ShapeDtypeStruct((_B, _N, _D), jnp.float32),
    )(adj, fs, mats, vecs, seproj)

    xst = xs.transpose(1, 0, 2)

    smats = jnp.stack([p['Wdt_0'], p['WB_0'], p['WC_0'],
                       p['Wdt_1'], p['WB_1'], p['WC_1'], p['Wr']])
    at0 = -jnp.exp(p['Alog_0']).T
    at1 = -jnp.exp(p['Alog_1']).T
    amats = jnp.stack([
        jnp.concatenate([at0[:_S // 2], at0[_S // 2:]], axis=1),
        jnp.concatenate([at1[:_S // 2], at1[_S // 2:]], axis=1)])
    svecs = jnp.stack([p['sln_g0'], p['sln_b0'], p['bdt_0'], p['Dp_0'],
                       p['sln_g1'], p['sln_b1'], p['bdt_1'], p['Dp_1'],
                       p['rln_g'], p['rln_b'], p['br']])

    col = jnp.arange(32 * 2 * _D)[None, :]
    m2 = (jnp.arange(_S)[:, None]
          == col // (2 * _D) + 32 * ((col % (2 * _D)) >= _D))
    m2 = m2.astype(jnp.bfloat16)

    out = pl.pallas_call(
        _ssm_body,
        in_specs=[
            pl.BlockSpec(xst.shape, lambda: (0, 0, 0)),
            pl.BlockSpec(smats.shape, lambda: (0, 0, 0)),
            pl.BlockSpec(amats.shape, lambda: (0, 0, 0)),
            pl.BlockSpec(svecs.shape, lambda: (0, 0)),
            pl.BlockSpec(m2.shape, lambda: (0, 0)),
        ],
        out_specs=pl.BlockSpec((_B, _D), lambda: (0, 0)),
        out_shape=jax.ShapeDtypeStruct((_B, _D), jnp.float32),
        scratch_shapes=[pltpu.VMEM((_N * _B, 2 * _D), jnp.float32),
                        pltpu.VMEM((_N * _B, 32, 2 * _D), jnp.bfloat16),
                        pltpu.VMEM((_N * _B, 32, 2 * _D), jnp.bfloat16),
                        pltpu.VMEM((_N * _B, 2 * _D), jnp.float32),
                        pltpu.VMEM((_N * _B, _D), jnp.float32)],
    )(xst, smats, amats, svecs, m2)
    return out
